# baseline (speedup 1.0000x reference)
import functools

import jax
import jax.numpy as jnp
from jax import lax
from jax.experimental import pallas as pl
from jax.experimental.pallas import tpu as pltpu
from jax.experimental.pallas import tpu_sc as plsc

_NUM_CORES = 2
_NUM_SUBCORES = 16
_NUM_WORKERS = _NUM_CORES * _NUM_SUBCORES

_CHUNK = 16
_NBUF = 6


@functools.partial(jax.jit, static_argnames=("batch", "seq", "hidden"))
def _lookup(positions, table, *, batch, seq, hidden):
    n_rows = batch * seq
    stripe = seq // _NUM_WORKERS
    chunks_per_row = stripe // _CHUNK
    n_chunks = batch * chunks_per_row
    mesh = plsc.VectorSubcoreMesh(core_axis_name="c", subcore_axis_name="s")

    @functools.partial(
        pl.kernel,
        mesh=mesh,
        out_type=jax.ShapeDtypeStruct((n_rows, hidden), jnp.float32),
        scratch_types=(
            [pltpu.VMEM((batch, stripe), jnp.int32)]
            + [pltpu.VMEM((_CHUNK, hidden), jnp.float32)] * _NBUF
            + [pltpu.SemaphoreType.DMA] * (2 * _NBUF)
        ),
    )
    def emb_kernel(idx_hbm, table_hbm, out_hbm, idx_v, *bufs):
        rows = bufs[:_NBUF]
        gsem = bufs[_NBUF:2 * _NBUF]
        osem = bufs[2 * _NBUF:]

        wid = lax.axis_index("s") * _NUM_CORES + lax.axis_index("c")
        col0 = wid * stripe

        pltpu.sync_copy(idx_hbm.at[:, pl.ds(col0, stripe)], idx_v)

        gcp = [None] * n_chunks
        ocp = [None] * n_chunks

        def out_off(g):
            r, c = divmod(g, chunks_per_row)
            return r * seq + col0 + c * _CHUNK

        def writeback(g):
            b = g % _NBUF
            gcp[g].wait()
            ocp[g] = pltpu.async_copy(
                rows[b], out_hbm.at[pl.ds(out_off(g), _CHUNK)], osem[b])

        for g in range(n_chunks):
            b = g % _NBUF
            r, c = divmod(g, chunks_per_row)
            if g >= _NBUF:
                ocp[g - _NBUF].wait()
            gcp[g] = pltpu.async_copy(
                table_hbm.at[idx_v.at[r, pl.ds(c * _CHUNK, _CHUNK)]],
                rows[b], gsem[b])
            if g >= 4:
                writeback(g - 4)

        for g in range(n_chunks - 4, n_chunks):
            writeback(g)
        for g in range(max(0, n_chunks - _NBUF), n_chunks):
            ocp[g].wait()

    return emb_kernel(positions, table)


def kernel(positions, pos_embedding):
    b, t = positions.shape
    hidden = pos_embedding.shape[1]
    out = _lookup(positions.astype(jnp.int32), pos_embedding,
                  batch=b, seq=t, hidden=hidden)
    return out.reshape(b, t, hidden)


# speedup vs baseline: 1.0060x; 1.0060x over previous
import functools

import jax
import jax.numpy as jnp
from jax import lax
from jax.experimental import pallas as pl
from jax.experimental.pallas import tpu as pltpu
from jax.experimental.pallas import tpu_sc as plsc

_NUM_CORES = 2
_NUM_SUBCORES = 16
_NUM_WORKERS = _NUM_CORES * _NUM_SUBCORES

_CHUNK = 32
_NBUF = 3


@functools.partial(jax.jit, static_argnames=("batch", "seq", "hidden"))
def _lookup(positions, table, *, batch, seq, hidden):
    n_rows = batch * seq
    stripe = seq // _NUM_WORKERS
    chunks_per_row = stripe // _CHUNK
    n_chunks = batch * chunks_per_row
    mesh = plsc.VectorSubcoreMesh(core_axis_name="c", subcore_axis_name="s")

    @functools.partial(
        pl.kernel,
        mesh=mesh,
        out_type=jax.ShapeDtypeStruct((n_rows, hidden), jnp.float32),
        scratch_types=(
            [pltpu.VMEM((batch, stripe), jnp.int32)]
            + [pltpu.VMEM((_CHUNK, hidden), jnp.float32)] * _NBUF
            + [pltpu.SemaphoreType.DMA] * (2 * _NBUF)
        ),
    )
    def emb_kernel(idx_hbm, table_hbm, out_hbm, idx_v, *bufs):
        rows = bufs[:_NBUF]
        gsem = bufs[_NBUF:2 * _NBUF]
        osem = bufs[2 * _NBUF:]

        wid = lax.axis_index("s") * _NUM_CORES + lax.axis_index("c")
        col0 = wid * stripe

        pltpu.sync_copy(idx_hbm.at[:, pl.ds(col0, stripe)], idx_v)

        gcp = [None] * n_chunks
        ocp = [None] * n_chunks

        def out_off(g):
            r, c = divmod(g, chunks_per_row)
            return r * seq + col0 + c * _CHUNK

        def writeback(g):
            b = g % _NBUF
            gcp[g].wait()
            ocp[g] = pltpu.async_copy(
                rows[b], out_hbm.at[pl.ds(out_off(g), _CHUNK)], osem[b])

        for g in range(n_chunks):
            b = g % _NBUF
            r, c = divmod(g, chunks_per_row)
            if g >= _NBUF:
                ocp[g - _NBUF].wait()
            gcp[g] = pltpu.async_copy(
                table_hbm.at[idx_v.at[r, pl.ds(c * _CHUNK, _CHUNK)]],
                rows[b], gsem[b])
            if g >= 2:
                writeback(g - 2)

        writeback(n_chunks - 2)
        writeback(n_chunks - 1)
        for g in range(max(0, n_chunks - _NBUF), n_chunks):
            ocp[g].wait()

    return emb_kernel(positions, table)


def kernel(positions, pos_embedding):
    b, t = positions.shape
    hidden = pos_embedding.shape[1]
    out = _lookup(positions.astype(jnp.int32), pos_embedding,
                  batch=b, seq=t, hidden=hidden)
    return out.reshape(b, t, hidden)
